# Initial kernel scaffold; baseline (speedup 1.0000x reference)
import jax
import jax.numpy as jnp
from jax.experimental import pallas as pl


def kernel(feats, adj, W1, b1, W2, b2, Wi, Wh, b_lstm):
    raise NotImplementedError("write your pallas kernel here")


# speedup vs baseline: 1.0318x; 1.0318x over previous
import functools

import jax
import jax.numpy as jnp
from jax.experimental import pallas as pl
from jax.experimental.pallas import tpu as pltpu

N = 4096
T = 4
DF = 128
NHID = 32
NCLASS = 16

BN = 1024
R = N // BN


def _body(adj_ref, xl_ref, W1_ref, b1_ref, W2_ref, b2_ref,
          Wi_ref, Wh_ref, bl_ref, out_ref, Y_s, G_s, H2_s):
    t = pl.program_id(0)
    p = pl.program_id(1)
    r = pl.program_id(2)

    @pl.when((t == 0) & (p == 0) & (r == 0))
    def _():
        Y_s[...] = jnp.dot(xl_ref[...], W1_ref[...],
                           preferred_element_type=jnp.float32)

    ab = adj_ref[0].astype(jnp.bfloat16)

    @pl.when(p == 0)
    def _():
        h1 = jnp.dot(ab, Y_s[...].astype(jnp.bfloat16),
                     preferred_element_type=jnp.float32) + b1_ref[...]
        h1 = jnp.maximum(h1, 0.0)
        G_s[pl.ds(r * BN, BN), :] = jnp.dot(
            h1, W2_ref[...], preferred_element_type=jnp.float32
        ).astype(jnp.bfloat16)

    @pl.when(p == 1)
    def _():
        h2 = jnp.dot(ab, G_s[...],
                     preferred_element_type=jnp.float32) + b2_ref[...]
        H2_s[pl.ds(t * N + r * BN, BN), :] = h2

    @pl.when((t == T - 1) & (p == 1) & (r == R - 1))
    def _():
        h = jnp.zeros((N, NCLASS), dtype=jnp.float32)
        c = jnp.zeros((N, NCLASS), dtype=jnp.float32)
        b = bl_ref[...]
        for step in range(T):
            x = H2_s[pl.ds(step * N, N), :]
            z = (jnp.dot(x, Wi_ref[...], preferred_element_type=jnp.float32)
                 + jnp.dot(h, Wh_ref[...], preferred_element_type=jnp.float32)
                 + b)
            i_g = jax.nn.sigmoid(z[:, :NCLASS])
            f_g = jax.nn.sigmoid(z[:, NCLASS:2 * NCLASS])
            g = jnp.tanh(z[:, 2 * NCLASS:3 * NCLASS])
            o_g = jax.nn.sigmoid(z[:, 3 * NCLASS:])
            c = f_g * c + i_g * g
            h = o_g * jnp.tanh(c)
        m = jnp.max(h, axis=1, keepdims=True)
        e = jnp.exp(h - m)
        out_ref[...] = e / jnp.sum(e, axis=1, keepdims=True)


@functools.partial(jax.jit, static_argnames=())
def kernel(feats, adj, W1, b1, W2, b2, Wi, Wh, b_lstm):
    x_last = feats[:, -1, :]
    b1r = b1.reshape(1, NHID)
    b2r = b2.reshape(1, NCLASS)
    blr = b_lstm.reshape(1, 4 * NCLASS)

    grid = (T, 2, R)
    out = pl.pallas_call(
        _body,
        grid=grid,
        in_specs=[
            pl.BlockSpec((1, BN, N), lambda t, p, r: (t, r, 0)),
            pl.BlockSpec((N, DF), lambda t, p, r: (0, 0)),
            pl.BlockSpec((DF, NHID), lambda t, p, r: (0, 0)),
            pl.BlockSpec((1, NHID), lambda t, p, r: (0, 0)),
            pl.BlockSpec((NHID, NCLASS), lambda t, p, r: (0, 0)),
            pl.BlockSpec((1, NCLASS), lambda t, p, r: (0, 0)),
            pl.BlockSpec((NCLASS, 4 * NCLASS), lambda t, p, r: (0, 0)),
            pl.BlockSpec((NCLASS, 4 * NCLASS), lambda t, p, r: (0, 0)),
            pl.BlockSpec((1, 4 * NCLASS), lambda t, p, r: (0, 0)),
        ],
        out_specs=pl.BlockSpec((N, NCLASS), lambda t, p, r: (0, 0)),
        out_shape=jax.ShapeDtypeStruct((N, NCLASS), jnp.float32),
        scratch_shapes=[
            pltpu.VMEM((N, NHID), jnp.float32),
            pltpu.VMEM((N, NCLASS), jnp.bfloat16),
            pltpu.VMEM((T * N, NCLASS), jnp.float32),
        ],
        compiler_params=pltpu.CompilerParams(
            vmem_limit_bytes=100 * 1024 * 1024,
        ),
    )(adj, x_last, W1, b1r, W2, b2r, Wi, Wh, blr)
    return out


# speedup vs baseline: 1.1442x; 1.1089x over previous
import jax
import jax.numpy as jnp
from jax.experimental import pallas as pl
from jax.experimental.pallas import tpu as pltpu

N = 4096
T = 4
DF = 128
NHID = 32
NCLASS = 16

PH = N // 2
CW = 512
NC = N // CW
STEPS_PER_T = 2 * NC
NSTEPS = T * STEPS_PER_T + 1


def _y_body(xl_ref, W1_ref, y_ref):
    y_ref[...] = jnp.dot(xl_ref[...], W1_ref[...],
                         preferred_element_type=jnp.float32
                         ).astype(jnp.bfloat16)


def _lstm_step(x, h, c, Wi_ref, Wh_ref, b):
    z = (jnp.dot(x, Wi_ref[...], preferred_element_type=jnp.float32)
         + jnp.dot(h, Wh_ref[...], preferred_element_type=jnp.float32)
         + b)
    i_g = jax.nn.sigmoid(z[:, :NCLASS])
    f_g = jax.nn.sigmoid(z[:, NCLASS:2 * NCLASS])
    g = jnp.tanh(z[:, 2 * NCLASS:3 * NCLASS])
    o_g = jax.nn.sigmoid(z[:, 3 * NCLASS:])
    c = f_g * c + i_g * g
    h = o_g * jnp.tanh(c)
    return h, c


def _body(adj_ref, Y_ref, b1_ref, W2_ref, b2_ref, Wi_ref, Wh_ref, bl_ref,
          out_ref, Ab_s, acc_s, G_s, o_s, h_s, c_s):
    s = pl.program_id(0)
    sc = jnp.minimum(s, T * STEPS_PER_T - 1)
    tt = sc // STEPS_PER_T
    rp = (sc // NC) % 2
    c = sc % NC

    @pl.when(s == 0)
    def _():
        h_s[...] = jnp.zeros_like(h_s)
        c_s[...] = jnp.zeros_like(c_s)
        o_s[...] = jnp.zeros_like(o_s)
        acc_s[...] = jnp.zeros_like(acc_s)

    @pl.when((s < NSTEPS - 1) & (tt >= 1) & (rp == 0) & (c <= NC - 2))
    def _():
        col = (c + 1) * CW
        o_s[...] += jnp.dot(Ab_s[:, pl.ds(col, CW)],
                            G_s[pl.ds(col, CW), :],
                            preferred_element_type=jnp.float32)

    @pl.when(s < NSTEPS - 1)
    def _():
        ab = adj_ref[0].astype(jnp.bfloat16)
        Ab_s[pl.ds(rp * PH, PH), pl.ds(c * CW, CW)] = ab
        yb = Y_ref[pl.ds(c * CW, CW), :]
        acc_s[pl.ds(rp * PH, PH), :] += jnp.dot(
            ab, yb, preferred_element_type=jnp.float32)

    @pl.when((s < NSTEPS - 1) & (rp == 1) & (c == NC - 1))
    def _():
        h1 = jnp.maximum(acc_s[...] + b1_ref[...], 0.0)
        G_s[...] = jnp.dot(h1, W2_ref[...],
                           preferred_element_type=jnp.float32
                           ).astype(jnp.bfloat16)
        acc_s[...] = jnp.zeros_like(acc_s)
        o_s[...] += jnp.dot(Ab_s[:, pl.ds(0, CW)], G_s[pl.ds(0, CW), :],
                            preferred_element_type=jnp.float32)

    @pl.when((s < NSTEPS - 1) & (tt >= 1) & (rp == 0) & (c == NC - 1))
    def _():
        x = o_s[...] + b2_ref[...]
        h, cst = _lstm_step(x, h_s[...], c_s[...], Wi_ref, Wh_ref,
                            bl_ref[...])
        h_s[...] = h
        c_s[...] = cst
        o_s[...] = jnp.zeros_like(o_s)

    @pl.when(s == NSTEPS - 1)
    def _():
        o_s[...] += jnp.dot(Ab_s[:, pl.ds(CW, N - CW)],
                            G_s[pl.ds(CW, N - CW), :],
                            preferred_element_type=jnp.float32)
        x = o_s[...] + b2_ref[...]
        h, _ = _lstm_step(x, h_s[...], c_s[...], Wi_ref, Wh_ref,
                          bl_ref[...])
        m = jnp.max(h, axis=1, keepdims=True)
        e = jnp.exp(h - m)
        out_ref[...] = e / jnp.sum(e, axis=1, keepdims=True)


def _adj_index(s):
    sc = jnp.minimum(s, T * STEPS_PER_T - 1)
    return (sc // STEPS_PER_T, (sc // NC) % 2, sc % NC)


def kernel(feats, adj, W1, b1, W2, b2, Wi, Wh, b_lstm):
    x_last = feats[:, -1, :]
    b1r = b1.reshape(1, NHID)
    b2r = b2.reshape(1, NCLASS)
    blr = b_lstm.reshape(1, 4 * NCLASS)

    Yb = pl.pallas_call(
        _y_body,
        out_shape=jax.ShapeDtypeStruct((N, NHID), jnp.bfloat16),
    )(x_last, W1)

    out = pl.pallas_call(
        _body,
        grid=(NSTEPS,),
        in_specs=[
            pl.BlockSpec((1, PH, CW), _adj_index),
            pl.BlockSpec((N, NHID), lambda s: (0, 0)),
            pl.BlockSpec((1, NHID), lambda s: (0, 0)),
            pl.BlockSpec((NHID, NCLASS), lambda s: (0, 0)),
            pl.BlockSpec((1, NCLASS), lambda s: (0, 0)),
            pl.BlockSpec((NCLASS, 4 * NCLASS), lambda s: (0, 0)),
            pl.BlockSpec((NCLASS, 4 * NCLASS), lambda s: (0, 0)),
            pl.BlockSpec((1, 4 * NCLASS), lambda s: (0, 0)),
        ],
        out_specs=pl.BlockSpec((N, NCLASS), lambda s: (0, 0)),
        out_shape=jax.ShapeDtypeStruct((N, NCLASS), jnp.float32),
        scratch_shapes=[
            pltpu.VMEM((N, N), jnp.bfloat16),
            pltpu.VMEM((N, NHID), jnp.float32),
            pltpu.VMEM((N, NCLASS), jnp.bfloat16),
            pltpu.VMEM((N, NCLASS), jnp.float32),
            pltpu.VMEM((N, NCLASS), jnp.float32),
            pltpu.VMEM((N, NCLASS), jnp.float32),
        ],
        compiler_params=pltpu.CompilerParams(
            vmem_limit_bytes=60 * 1024 * 1024,
        ),
    )(adj, Yb, b1r, W2, b2r, Wi, Wh, blr)
    return out
